# baseline (speedup 1.0000x reference)
import functools

import numpy as np
import jax
import jax.numpy as jnp
from jax import lax
from jax.experimental import pallas as pl
from jax.experimental.pallas import tpu as pltpu
from jax.experimental.pallas import tpu_sc as plsc

_N = 100000
_D = 128
_B = 4096

_TILE_N = 2048
_NPAD = 100352
_NSTEPS = _NPAD // _TILE_N
_R = 8
_NB = _B // _R

_K0 = np.uint32(2247515013)
_K1 = np.uint32(2545468385)
_KS2 = np.uint32(_K0 ^ _K1 ^ np.uint32(0x1BD11BDA))

_TINY = np.float32(np.finfo(np.float32).tiny)
_ROT_A = (13, 15, 26, 6)
_ROT_B = (17, 29, 16, 24)


def _threefry_bits(p):
    ks = (jnp.uint32(_K0), jnp.uint32(_K1), jnp.uint32(_KS2))
    x0 = jnp.full(p.shape, _K0, dtype=jnp.uint32)
    x1 = p + ks[1]
    for i in range(1, 6):
        rots = _ROT_A if i % 2 == 1 else _ROT_B
        for r in rots:
            x0 = x0 + x1
            x1 = (x1 << jnp.uint32(r)) | (x1 >> jnp.uint32(32 - r))
            x1 = x1 ^ x0
        x0 = x0 + ks[i % 3]
        x1 = x1 + ks[(i + 1) % 3] + jnp.uint32(i)
    return x0 ^ x1


def _gumbel_from_bits(bits):
    fb = (bits >> jnp.uint32(9)) | jnp.uint32(0x3F800000)
    f = lax.bitcast_convert_type(fb, jnp.float32) - jnp.float32(1.0)
    u = jnp.maximum(f, _TINY)
    return -jnp.log(-jnp.log(u))


def _sampler_body(w_ref, lse_ref, idx_ref, logw_ref):
    b0 = pl.program_id(0)
    row = jax.lax.broadcasted_iota(jnp.uint32, (_R, _TILE_N), 0)
    col = jax.lax.broadcasted_iota(jnp.uint32, (_R, _TILE_N), 1)
    pbase = (jnp.uint32(b0) * jnp.uint32(_R) + row) * jnp.uint32(_N) + col
    coli = jax.lax.broadcasted_iota(jnp.int32, (_R, _TILE_N), 1)

    def step(t, carry):
        bv, bn, bw = carry
        n0 = t * _TILE_N
        p = pbase + jnp.uint32(n0).astype(jnp.uint32)
        g = _gumbel_from_bits(_threefry_bits(p))
        wt = w_ref[0:1, pl.ds(n0, _TILE_N)]
        c = g + wt
        mask = c > bv
        bv = jnp.where(mask, c, bv)
        bn = jnp.where(mask, coli + n0, bn)
        bw = jnp.where(mask, jnp.broadcast_to(wt, (_R, _TILE_N)), bw)
        return bv, bn, bw

    init = (
        jnp.full((_R, _TILE_N), -jnp.inf, dtype=jnp.float32),
        jnp.zeros((_R, _TILE_N), dtype=jnp.int32),
        jnp.full((_R, _TILE_N), -jnp.inf, dtype=jnp.float32),
    )
    bv, bn, bw = lax.fori_loop(0, _NSTEPS, step, init)

    m = jnp.max(bv, axis=1, keepdims=True)
    big = jnp.int32(np.int32(2**31 - 1))
    idx = jnp.min(jnp.where(bv == m, bn, big), axis=1, keepdims=True)
    wsel = jnp.max(jnp.where(bn == idx, bw, -jnp.inf), axis=1, keepdims=True)
    idx_ref[0, :, :] = idx
    logw_ref[0, :, :] = wsel - lse_ref[0, 0]


def _lse_body(w_ref, out_ref):
    w = w_ref[...]
    m = jnp.max(w)
    s = jnp.sum(jnp.exp(w - m))
    out_ref[...] = jnp.broadcast_to(m + jnp.log(s), (1, 1))


_NC = 2
_NS = 16
_NW = _NC * _NS
_BPW = _B // _NW


def _gather_body(table_ref, idx_ref, out_ref, idx_v, rows_v, sem):
    wid = lax.axis_index("s") * _NC + lax.axis_index("c")
    base = wid * _BPW
    pltpu.sync_copy(idx_ref.at[pl.ds(base, _BPW)], idx_v)
    pltpu.async_copy(table_ref.at[idx_v], rows_v, sem).wait()
    pltpu.sync_copy(rows_v, out_ref.at[pl.ds(base, _BPW)])


def kernel(data, weights):
    w_pad = jnp.pad(
        weights.reshape(1, _N), ((0, 0), (0, _NPAD - _N)),
        constant_values=-np.inf)

    lse = pl.pallas_call(
        _lse_body,
        out_shape=jax.ShapeDtypeStruct((1, 1), jnp.float32),
        in_specs=[pl.BlockSpec((1, _NPAD), lambda: (0, 0))],
        out_specs=pl.BlockSpec((1, 1), lambda: (0, 0)),
    )(w_pad)

    idx3, logw3 = pl.pallas_call(
        _sampler_body,
        grid=(_NB,),
        out_shape=(
            jax.ShapeDtypeStruct((_NB, _R, 1), jnp.int32),
            jax.ShapeDtypeStruct((_NB, _R, 1), jnp.float32),
        ),
        in_specs=[
            pl.BlockSpec((1, _NPAD), lambda i: (0, 0)),
            pl.BlockSpec((1, 1), lambda i: (0, 0)),
        ],
        out_specs=(
            pl.BlockSpec((1, _R, 1), lambda i: (i, 0, 0)),
            pl.BlockSpec((1, _R, 1), lambda i: (i, 0, 0)),
        ),
        compiler_params=pltpu.CompilerParams(
            dimension_semantics=("parallel",)),
    )(w_pad, lse)

    indices = idx3.reshape(_B)
    logw = logw3.reshape(_B)

    mesh = plsc.VectorSubcoreMesh(core_axis_name="c", subcore_axis_name="s")
    gather = pl.kernel(
        _gather_body,
        out_type=jax.ShapeDtypeStruct((_B, _D), jnp.float32),
        mesh=mesh,
        scratch_types=[
            pltpu.VMEM((_BPW,), jnp.int32),
            pltpu.VMEM((_BPW, _D), jnp.float32),
            pltpu.SemaphoreType.DMA,
        ],
    )
    batch = gather(data, indices)
    return (batch, logw, indices)


# speedup vs baseline: 1.1057x; 1.1057x over previous
import numpy as np
import jax
import jax.numpy as jnp
from jax import lax
from jax.experimental import pallas as pl
from jax.experimental.pallas import tpu as pltpu
from jax.experimental.pallas import tpu_sc as plsc

_N = 100000
_D = 128
_B = 4096

_TILE_N = 2048
_NPAD = 100352
_NSTEPS = _NPAD // _TILE_N
_R = 8
_NB = _B // _R

_K0 = np.uint32(2247515013)
_K1 = np.uint32(2545468385)
_KS2 = np.uint32(_K0 ^ _K1 ^ np.uint32(0x1BD11BDA))
_KS = (int(_K0), int(_K1), int(_KS2))
_INJ = tuple(
    (np.uint32(_KS[i % 3]), np.uint32((_KS[(i + 1) % 3] + i) & 0xFFFFFFFF))
    for i in range(1, 6))

_TINY = np.float32(np.finfo(np.float32).tiny)
_ROT_A = (13, 15, 26, 6)
_ROT_B = (17, 29, 16, 24)


def _threefry_bits(x1):
    x0 = jnp.full(x1.shape, _K0, dtype=jnp.uint32)
    for i, (c0, c1) in enumerate(_INJ):
        rots = _ROT_A if i % 2 == 0 else _ROT_B
        for r in rots:
            x0 = x0 + x1
            x1 = (x1 << jnp.uint32(r)) | (x1 >> jnp.uint32(32 - r))
            x1 = x1 ^ x0
        x0 = x0 + c0
        x1 = x1 + c1
    return x0 ^ x1


def _gumbel_from_bits(bits):
    fb = (bits >> jnp.uint32(9)) | jnp.uint32(0x3F800000)
    f = lax.bitcast_convert_type(fb, jnp.float32) - jnp.float32(1.0)
    u = jnp.maximum(f, _TINY)
    return -jnp.log(-jnp.log(u))


def _sampler_body(w_ref, idx_ref):
    b0 = pl.program_id(0)
    row = lax.broadcasted_iota(jnp.uint32, (_R, _TILE_N), 0)
    col = lax.broadcasted_iota(jnp.uint32, (_R, _TILE_N), 1)
    px1 = ((jnp.uint32(b0) * jnp.uint32(_R) + row) * jnp.uint32(_N)
           + col + jnp.uint32(_K1))

    def step(t, carry):
        bv, bt = carry
        n0 = t * _TILE_N
        g = _gumbel_from_bits(_threefry_bits(px1 + n0.astype(jnp.uint32)))
        wt = w_ref[0:1, pl.ds(n0, _TILE_N)]
        c = g + wt
        mask = c > bv
        bv = jnp.where(mask, c, bv)
        bt = jnp.where(mask, t, bt)
        return bv, bt

    init = (
        jnp.full((_R, _TILE_N), -jnp.inf, dtype=jnp.float32),
        jnp.zeros((_R, _TILE_N), dtype=jnp.int32),
    )
    bv, bt = lax.fori_loop(0, _NSTEPS, step, init, unroll=2)

    coli = lax.broadcasted_iota(jnp.int32, (_R, _TILE_N), 1)
    ncand = bt * _TILE_N + coli
    m = jnp.max(bv, axis=1, keepdims=True)
    big = jnp.int32(np.int32(2**31 - 1))
    idx = jnp.min(jnp.where(bv == m, ncand, big), axis=1, keepdims=True)
    idx_ref[0, :, :] = idx


def _lse_body(w_ref, out_ref):
    w = w_ref[...]
    m = jnp.max(w)
    s = jnp.sum(jnp.exp(w - m))
    out_ref[...] = jnp.broadcast_to(m + jnp.log(s), (1, 16))


_NC = 2
_NS = 16
_NW = _NC * _NS
_BPW = _B // _NW


def _gather_body(table_ref, w_hbm_ref, lse_ref, idx_ref,
                 out_ref, logw_ref,
                 idx_v, rows_v, wv, logw_v, lse_v, sem, sem2):
    wid = lax.axis_index("s") * _NC + lax.axis_index("c")
    base = wid * _BPW
    pltpu.sync_copy(idx_ref.at[pl.ds(base, _BPW)], idx_v)
    c1 = pltpu.async_copy(table_ref.at[idx_v], rows_v, sem)
    c2 = pltpu.async_copy(w_hbm_ref.at[idx_v], wv, sem2)
    pltpu.sync_copy(lse_ref, lse_v)
    c2.wait()
    lv = lse_v[...]
    for j in range(_BPW // 16):
        logw_v[pl.ds(j * 16, 16)] = wv[pl.ds(j * 16, 16)] - lv
    c1.wait()
    pltpu.sync_copy(rows_v, out_ref.at[pl.ds(base, _BPW)])
    pltpu.sync_copy(logw_v, logw_ref.at[pl.ds(base, _BPW)])


def kernel(data, weights):
    w_pad = jnp.pad(
        weights.reshape(1, _N), ((0, 0), (0, _NPAD - _N)),
        constant_values=-np.inf)

    lse = pl.pallas_call(
        _lse_body,
        out_shape=jax.ShapeDtypeStruct((1, 16), jnp.float32),
        in_specs=[pl.BlockSpec((1, _NPAD), lambda: (0, 0))],
        out_specs=pl.BlockSpec((1, 16), lambda: (0, 0)),
    )(w_pad)

    idx3 = pl.pallas_call(
        _sampler_body,
        grid=(_NB,),
        out_shape=jax.ShapeDtypeStruct((_NB, _R, 1), jnp.int32),
        in_specs=[pl.BlockSpec((1, _NPAD), lambda i: (0, 0))],
        out_specs=pl.BlockSpec((1, _R, 1), lambda i: (i, 0, 0)),
        compiler_params=pltpu.CompilerParams(
            dimension_semantics=("parallel",)),
    )(w_pad)

    indices = idx3.reshape(_B)

    mesh = plsc.VectorSubcoreMesh(core_axis_name="c", subcore_axis_name="s")
    gather = pl.kernel(
        _gather_body,
        out_type=(
            jax.ShapeDtypeStruct((_B, _D), jnp.float32),
            jax.ShapeDtypeStruct((_B,), jnp.float32),
        ),
        mesh=mesh,
        scratch_types=[
            pltpu.VMEM((_BPW,), jnp.int32),
            pltpu.VMEM((_BPW, _D), jnp.float32),
            pltpu.VMEM((_BPW,), jnp.float32),
            pltpu.VMEM((_BPW,), jnp.float32),
            pltpu.VMEM((16,), jnp.float32),
            pltpu.SemaphoreType.DMA,
            pltpu.SemaphoreType.DMA,
        ],
    )
    batch, logw = gather(data, weights, lse.reshape(16), indices)
    return (batch, logw, indices)


# speedup vs baseline: 1.1082x; 1.0023x over previous
import numpy as np
import jax
import jax.numpy as jnp
from jax import lax
from jax.experimental import pallas as pl
from jax.experimental.pallas import tpu as pltpu
from jax.experimental.pallas import tpu_sc as plsc

_N = 100000
_D = 128
_B = 4096

_TILE_N = 2048
_NPAD = 100352
_NSTEPS = _NPAD // _TILE_N
_R = 8
_NB = _B // _R

_K0 = np.uint32(2247515013)
_K1 = np.uint32(2545468385)
_KS2 = np.uint32(_K0 ^ _K1 ^ np.uint32(0x1BD11BDA))
_KS = (int(_K0), int(_K1), int(_KS2))
_INJ = tuple(
    (np.uint32(_KS[i % 3]), np.uint32((_KS[(i + 1) % 3] + i) & 0xFFFFFFFF))
    for i in range(1, 6))

_TINY = np.float32(np.finfo(np.float32).tiny)
_ROT_A = (13, 15, 26, 6)
_ROT_B = (17, 29, 16, 24)


def _threefry_bits(x1):
    x0 = jnp.full(x1.shape, _K0, dtype=jnp.uint32)
    for i, (c0, c1) in enumerate(_INJ):
        rots = _ROT_A if i % 2 == 0 else _ROT_B
        for r in rots:
            x0 = x0 + x1
            x1 = (x1 << jnp.uint32(r)) | (x1 >> jnp.uint32(32 - r))
            x1 = x1 ^ x0
        x0 = x0 + c0
        x1 = x1 + c1
    return x0 ^ x1


def _gumbel_from_bits(bits):
    fb = (bits >> jnp.uint32(9)) | jnp.uint32(0x3F800000)
    f = lax.bitcast_convert_type(fb, jnp.float32) - jnp.float32(1.0)
    u = jnp.maximum(f, _TINY)
    return -jnp.log(-jnp.log(u))


def _sampler_body(w_ref, idx_ref):
    b0 = pl.program_id(0)
    row = lax.broadcasted_iota(jnp.uint32, (_R, _TILE_N), 0)
    col = lax.broadcasted_iota(jnp.uint32, (_R, _TILE_N), 1)
    px1 = ((jnp.uint32(b0) * jnp.uint32(_R) + row) * jnp.uint32(_N)
           + col + jnp.uint32(_K1))

    def step(t, carry):
        bv, bt = carry
        n0 = t * _TILE_N
        g = _gumbel_from_bits(_threefry_bits(px1 + n0.astype(jnp.uint32)))
        wt = w_ref[0:1, pl.ds(n0, _TILE_N)]
        c = g + wt
        mask = c > bv
        bv = jnp.where(mask, c, bv)
        bt = jnp.where(mask, t, bt)
        return bv, bt

    init = (
        jnp.full((_R, _TILE_N), -jnp.inf, dtype=jnp.float32),
        jnp.zeros((_R, _TILE_N), dtype=jnp.int32),
    )
    bv, bt = lax.fori_loop(0, _NSTEPS, step, init, unroll=7)

    coli = lax.broadcasted_iota(jnp.int32, (_R, _TILE_N), 1)
    ncand = bt * _TILE_N + coli
    m = jnp.max(bv, axis=1, keepdims=True)
    big = jnp.int32(np.int32(2**31 - 1))
    idx = jnp.min(jnp.where(bv == m, ncand, big), axis=1, keepdims=True)
    idx_ref[0, :, :] = idx


def _lse_body(w_ref, out_ref):
    w = w_ref[...]
    m = jnp.max(w)
    s = jnp.sum(jnp.exp(w - m))
    out_ref[...] = jnp.broadcast_to(m + jnp.log(s), (1, 16))


_NC = 2
_NS = 16
_NW = _NC * _NS
_BPW = _B // _NW


def _gather_body(table_ref, w_hbm_ref, lse_ref, idx_ref,
                 out_ref, logw_ref,
                 idx_v, rows_v, wv, logw_v, lse_v, sem, sem2):
    wid = lax.axis_index("s") * _NC + lax.axis_index("c")
    base = wid * _BPW
    pltpu.sync_copy(idx_ref.at[pl.ds(base, _BPW)], idx_v)
    c1 = pltpu.async_copy(table_ref.at[idx_v], rows_v, sem)
    c2 = pltpu.async_copy(w_hbm_ref.at[idx_v], wv, sem2)
    pltpu.sync_copy(lse_ref, lse_v)
    c2.wait()
    lv = lse_v[...]
    for j in range(_BPW // 16):
        logw_v[pl.ds(j * 16, 16)] = wv[pl.ds(j * 16, 16)] - lv
    c1.wait()
    pltpu.sync_copy(rows_v, out_ref.at[pl.ds(base, _BPW)])
    pltpu.sync_copy(logw_v, logw_ref.at[pl.ds(base, _BPW)])


def kernel(data, weights):
    w_pad = jnp.pad(
        weights.reshape(1, _N), ((0, 0), (0, _NPAD - _N)),
        constant_values=-np.inf)

    lse = pl.pallas_call(
        _lse_body,
        out_shape=jax.ShapeDtypeStruct((1, 16), jnp.float32),
        in_specs=[pl.BlockSpec((1, _NPAD), lambda: (0, 0))],
        out_specs=pl.BlockSpec((1, 16), lambda: (0, 0)),
    )(w_pad)

    idx3 = pl.pallas_call(
        _sampler_body,
        grid=(_NB,),
        out_shape=jax.ShapeDtypeStruct((_NB, _R, 1), jnp.int32),
        in_specs=[pl.BlockSpec((1, _NPAD), lambda i: (0, 0))],
        out_specs=pl.BlockSpec((1, _R, 1), lambda i: (i, 0, 0)),
        compiler_params=pltpu.CompilerParams(
            dimension_semantics=("parallel",)),
    )(w_pad)

    indices = idx3.reshape(_B)

    mesh = plsc.VectorSubcoreMesh(core_axis_name="c", subcore_axis_name="s")
    gather = pl.kernel(
        _gather_body,
        out_type=(
            jax.ShapeDtypeStruct((_B, _D), jnp.float32),
            jax.ShapeDtypeStruct((_B,), jnp.float32),
        ),
        mesh=mesh,
        scratch_types=[
            pltpu.VMEM((_BPW,), jnp.int32),
            pltpu.VMEM((_BPW, _D), jnp.float32),
            pltpu.VMEM((_BPW,), jnp.float32),
            pltpu.VMEM((_BPW,), jnp.float32),
            pltpu.VMEM((16,), jnp.float32),
            pltpu.SemaphoreType.DMA,
            pltpu.SemaphoreType.DMA,
        ],
    )
    batch, logw = gather(data, weights, lse.reshape(16), indices)
    return (batch, logw, indices)
